# baseline (speedup 1.0000x reference)
import functools
import numpy as np
import jax
import jax.numpy as jnp
from jax.experimental import pallas as pl
from jax.experimental.pallas import tpu as pltpu

B, N, H, E = 16, 325, 64, 5200
TAU = 0.1
G3 = 3 * H
EP = 5376
TE = 896
NCHUNK = EP // TE

_DOT = functools.partial(jnp.dot, preferred_element_type=jnp.float32,
                         precision=jax.lax.Precision.HIGHEST)


def _dg(a, b, ca, cb):
    return jax.lax.dot_general(
        a, b, (((ca,), (cb,)), ((), ())),
        preferred_element_type=jnp.float32,
        precision=jax.lax.Precision.HIGHEST)


def _gelu(x):
    return 0.5 * x * (1.0 + jax.lax.erf(x * np.float32(1.0 / np.sqrt(2.0))))


def _comb(gi, gh, hprev):
    r = jax.nn.sigmoid(gi[:, :H] + gh[:, :H])
    z = jax.nn.sigmoid(gi[:, H:2 * H] + gh[:, H:2 * H])
    n = jnp.tanh(gi[:, 2 * H:] + r * gh[:, 2 * H:])
    out = (1.0 - z) * n
    if hprev is not None:
        out = out + z * hprev
    return out


def _body(h_ref, dst_ref, src_ref, g_ref,
          wn_ref, bhf0_ref, bhr0_ref,
          w1_ref, b1_ref, w2_ref, b2_ref, w3_ref, b3_ref, w4_ref, b4_ref,
          w5_ref, w6_ref, w7_ref, b7_ref,
          bir1_ref, bif1_ref, bhr1_ref, bhf1_ref,
          wcf_ref, wcr_ref, bc_ref,
          out_ref):
    h = h_ref[0]
    dst = dst_ref[...]
    src = src_ref[...]
    g = g_ref[0]

    gnode = _DOT(h, wn_ref[...])
    gf = gnode[:, :G3] + bhf0_ref[0:1]
    gr = gnode[:, G3:] + bhr0_ref[0:1]
    hf1 = _comb(gf, jnp.broadcast_to(bhf0_ref[1:2], (N, G3)), None)
    hr2 = _comb(gr, jnp.broadcast_to(bhr0_ref[1:2], (N, G3)), None)

    logit_parts = []
    hs_parts = []
    for c in range(NCHUNK):
        sl = slice(c * TE, (c + 1) * TE)
        dst_c = dst[:, sl]
        src_c = src[:, sl]
        iota_c = jax.lax.broadcasted_iota(jnp.int32, (N, TE), 0)
        sd_c = (iota_c == dst_c).astype(jnp.float32)
        ss_c = (iota_c == src_c).astype(jnp.float32)
        hd = _dg(sd_c, h, 0, 0)
        f1d = _dg(sd_c, hf1, 0, 0)
        hs = _dg(ss_c, h, 0, 0)
        r2s = _dg(ss_c, hr2, 0, 0)

        m1 = _DOT(hd, w1_ref[...]) + b1_ref[...]
        m2 = _DOT(r2s, w2_ref[...]) + b2_ref[...]
        hr1 = _comb(m1, m2[:, :G3], r2s)
        m3 = _DOT(hs, w3_ref[...]) + b3_ref[...]
        m4 = _DOT(f1d, w4_ref[...]) + b4_ref[...]
        hf2 = _comb(m3, m4[:, :G3], f1d)
        m5 = _DOT(hf2, w5_ref[...])
        gi1 = m5 + m2[:, G3:] + bir1_ref[...]
        hr2l1 = _comb(gi1, jnp.broadcast_to(bhr1_ref[...], (TE, G3)), None)
        m6 = _DOT(hr1, w6_ref[...])
        m7 = _DOT(hr2l1, w7_ref[...]) + b7_ref[...]
        gi0r = m4[:, G3:2 * G3] + m6[:, :G3] + bir1_ref[...]
        o_r = _comb(gi0r, m7, hr2l1)
        gi0f = m4[:, 2 * G3:] + m6[:, G3:] + bif1_ref[...]
        o_f = _comb(gi0f, jnp.broadcast_to(bhf1_ref[...], (TE, G3)), None)

        x = (_dg(wcf_ref[...], o_f, 1, 1) + _dg(wcr_ref[...], o_r, 1, 1)
             + bc_ref[0, 0])
        logit_parts.append(_gelu(x))
        hs_parts.append(hs)

    logit = jnp.concatenate(logit_parts, axis=1)
    lanes = jax.lax.broadcasted_iota(jnp.int32, (1, EP), 1)
    valid = lanes < E
    logit = jnp.where(valid, logit, np.float32(-1e30))

    iota_full = jax.lax.broadcasted_iota(jnp.int32, (N, EP), 0)
    bs = iota_full == src
    ss = bs.astype(jnp.float32)
    zh = (logit + g) * np.float32(1.0 / TAU)
    zmask = jnp.where(bs, zh, np.float32(-1e30))
    m = jnp.max(zmask, axis=1, keepdims=True)
    msrc = _dg(m, ss, 0, 0)
    eh = jnp.exp(zh - msrc)
    eh = jnp.where(valid, eh, 0.0)
    shard = _dg(eh, ss, 1, 1)
    hard = eh / (_dg(shard, ss, 1, 0) + np.float32(1e-12))
    es = jnp.exp(logit)
    es = jnp.where(valid, es, 0.0)
    ssoft = _dg(es, ss, 1, 1)
    soft = es / (_dg(ssoft, ss, 1, 0) + np.float32(1e-12))
    coef = soft * hard
    coef = jnp.where(valid, coef, 0.0)

    acc = jnp.zeros((N, H), jnp.float32)
    for c in range(NCHUNK):
        sl = slice(c * TE, (c + 1) * TE)
        dst_c = dst[:, sl]
        iota_c = jax.lax.broadcasted_iota(jnp.int32, (N, TE), 0)
        sd_c = (iota_c == dst_c).astype(jnp.float32) * coef[:, sl]
        acc = acc + _dg(sd_c, hs_parts[c], 1, 0)
    out_ref[0] = acc


def kernel(h, params, edge_index):
    f32 = jnp.float32
    p0, p1 = params['l0'], params['l1']
    a1 = p1['W_ih_r'][:, :H]
    a2 = p1['W_ih_r'][:, H:]
    f1 = p1['W_ih_f'][:, :H]
    f2 = p1['W_ih_f'][:, H:]

    wn = jnp.concatenate([p0['W_ih_f'].T, p0['W_ih_r'].T], axis=1)
    bhf0 = jnp.stack([p0['b_ih_f'], p0['b_hh_f']], axis=0)
    bhr0 = jnp.stack([p0['b_ih_r'], p0['b_hh_r']], axis=0)

    w1 = p0['W_ih_r'].T
    b1 = p0['b_ih_r'][None, :]
    w2 = jnp.concatenate([p0['W_hh_r'].T, a2.T], axis=1)
    b2 = jnp.concatenate([p0['b_hh_r'], jnp.zeros((G3,), f32)])[None, :]
    w3 = p0['W_ih_f'].T
    b3 = p0['b_ih_f'][None, :]
    w4 = jnp.concatenate([p0['W_hh_f'].T, a1.T, f1.T], axis=1)
    b4 = jnp.concatenate([p0['b_hh_f'], jnp.zeros((2 * G3,), f32)])[None, :]
    w5 = a1.T
    w6 = jnp.concatenate([a2.T, f2.T], axis=1)
    w7 = p1['W_hh_r'].T
    b7 = p1['b_hh_r'][None, :]
    bir1 = p1['b_ih_r'][None, :]
    bif1 = p1['b_ih_f'][None, :]
    bhr1 = p1['b_hh_r'][None, :]
    bhf1 = p1['b_hh_f'][None, :]
    wcf = params['Wc'][0:1, :H]
    wcr = params['Wc'][0:1, H:]
    bc = params['bc'][None, :]

    u = jax.random.uniform(jax.random.key(42), (E, B),
                           minval=1e-6, maxval=1.0 - 1e-6)
    g = -jnp.log(-jnp.log(u))
    gp = jnp.zeros((B, 1, EP), f32).at[:, 0, :E].set(jnp.transpose(g))

    pad = jnp.full((1, EP - E), N, jnp.int32)
    dstr = jnp.concatenate([edge_index[0][None, :], pad], axis=1)
    srcr = jnp.concatenate([edge_index[1][None, :], pad], axis=1)

    full = lambda shape: pl.BlockSpec(shape, lambda b: (0,) * len(shape))
    grid_spec = pl.GridSpec(
        grid=(B,),
        in_specs=[
            pl.BlockSpec((1, N, H), lambda b: (b, 0, 0)),
            full((1, EP)), full((1, EP)),
            pl.BlockSpec((1, 1, EP), lambda b: (b, 0, 0)),
            full((H, 2 * G3)), full((2, G3)), full((2, G3)),
            full((H, G3)), full((1, G3)),
            full((H, 2 * G3)), full((1, 2 * G3)),
            full((H, G3)), full((1, G3)),
            full((H, 3 * G3)), full((1, 3 * G3)),
            full((H, G3)),
            full((H, 2 * G3)),
            full((H, G3)), full((1, G3)),
            full((1, G3)), full((1, G3)), full((1, G3)), full((1, G3)),
            full((1, H)), full((1, H)), full((1, 1)),
        ],
        out_specs=pl.BlockSpec((1, N, H), lambda b: (b, 0, 0)),
    )
    out = pl.pallas_call(
        _body,
        grid_spec=grid_spec,
        out_shape=jax.ShapeDtypeStruct((B, N, H), f32),
    )(h, dstr, srcr, gp,
      wn, bhf0, bhr0,
      w1, b1, w2, b2, w3, b3, w4, b4,
      w5, w6, w7, b7,
      bir1, bif1, bhr1, bhf1,
      wcf, wcr, bc)
    return out


# speedup vs baseline: 2.3174x; 2.3174x over previous
import numpy as np
import jax
import jax.numpy as jnp
from jax.experimental import pallas as pl
from jax.experimental.pallas import tpu as pltpu

B, N, H, E = 16, 325, 64, 5200
TAU = 0.1
G3 = 3 * H
EP = 5376
TE = 896
NCHUNK = EP // TE
BF = jnp.bfloat16
F32 = jnp.float32


def _sp(a):
    ah = a.astype(BF)
    al = (a - ah.astype(F32)).astype(BF)
    return ah, al


def _dgb(a, b, ca, cb):
    return jax.lax.dot_general(
        a, b, (((ca,), (cb,)), ((), ())), preferred_element_type=F32)


def _dot3(asp, bsp, ca=1, cb=0):
    ah, al = asp
    bh, bl = bsp
    return (_dgb(ah, bh, ca, cb) + _dgb(ah, bl, ca, cb)
            + _dgb(al, bh, ca, cb))


def _dot2(mask_bf, bsp, ca, cb):
    bh, bl = bsp
    return _dgb(mask_bf, bh, ca, cb) + _dgb(mask_bf, bl, ca, cb)


def _gelu(x):
    return 0.5 * x * (1.0 + jax.lax.erf(x * np.float32(1.0 / np.sqrt(2.0))))


def _comb(gi, gh, hprev):
    r = jax.nn.sigmoid(gi[:, :H] + gh[:, :H])
    z = jax.nn.sigmoid(gi[:, H:2 * H] + gh[:, H:2 * H])
    n = jnp.tanh(gi[:, 2 * H:] + r * gh[:, 2 * H:])
    out = (1.0 - z) * n
    if hprev is not None:
        out = out + z * hprev
    return out


def _body(h_ref, dst_ref, src_ref, g_ref,
          wn_ref, bhf0_ref, bhr0_ref,
          w1_ref, b1_ref, w2_ref, b2_ref, w3_ref, b3_ref, w4_ref, b4_ref,
          w5_ref, w6_ref, w7_ref, b7_ref,
          bir1_ref, bif1_ref, bhr1_ref, bhf1_ref,
          wcf_ref, wcr_ref, bc_ref,
          out_ref):
    h = h_ref[0]
    dst = dst_ref[...]
    src = src_ref[...]
    g = g_ref[0]

    wsp = lambda ref: (ref[0], ref[1])

    hsp = _sp(h)
    gnode = _dot3(hsp, wsp(wn_ref))
    gf = gnode[:, :G3] + bhf0_ref[0:1]
    gr = gnode[:, G3:] + bhr0_ref[0:1]
    hf1 = _comb(gf, jnp.broadcast_to(bhf0_ref[1:2], (N, G3)), None)
    hr2 = _comb(gr, jnp.broadcast_to(bhr0_ref[1:2], (N, G3)), None)

    td = _sp(jnp.concatenate([h, hf1], axis=1))
    ts = _sp(jnp.concatenate([h, hr2], axis=1))

    logit_parts = []
    hs_parts = []
    for c in range(NCHUNK):
        sl = slice(c * TE, (c + 1) * TE)
        dst_c = dst[:, sl]
        src_c = src[:, sl]
        iota_c = jax.lax.broadcasted_iota(jnp.int32, (N, TE), 0)
        sd_c = (iota_c == dst_c).astype(BF)
        ss_c = (iota_c == src_c).astype(BF)
        gd = _dot2(sd_c, td, 0, 0)
        gs = _dot2(ss_c, ts, 0, 0)
        hd = _sp(gd[:, :H])
        f1d_f = gd[:, H:]
        f1d = _sp(f1d_f)
        hs_f = gs[:, :H]
        hs = _sp(hs_f)
        r2s_f = gs[:, H:]
        r2s = _sp(r2s_f)

        m1 = _dot3(hd, wsp(w1_ref)) + b1_ref[...]
        m2 = _dot3(r2s, wsp(w2_ref)) + b2_ref[...]
        hr1 = _sp(_comb(m1, m2[:, :G3], r2s_f))
        m3 = _dot3(hs, wsp(w3_ref)) + b3_ref[...]
        m4 = _dot3(f1d, wsp(w4_ref)) + b4_ref[...]
        hf2 = _sp(_comb(m3, m4[:, :G3], f1d_f))
        m5 = _dot3(hf2, wsp(w5_ref))
        gi1 = m5 + m2[:, G3:] + bir1_ref[...]
        hr2l1_f = _comb(gi1, jnp.broadcast_to(bhr1_ref[...], (TE, G3)), None)
        hr2l1 = _sp(hr2l1_f)
        m6 = _dot3(hr1, wsp(w6_ref))
        m7 = _dot3(hr2l1, wsp(w7_ref)) + b7_ref[...]
        gi0r = m4[:, G3:2 * G3] + m6[:, :G3] + bir1_ref[...]
        o_r = _sp(_comb(gi0r, m7, hr2l1_f))
        gi0f = m4[:, 2 * G3:] + m6[:, G3:] + bif1_ref[...]
        o_f = _sp(_comb(gi0f, jnp.broadcast_to(bhf1_ref[...], (TE, G3)),
                        None))

        x = (_dot3(wsp(wcf_ref), o_f, 1, 1) + _dot3(wsp(wcr_ref), o_r, 1, 1)
             + bc_ref[0, 0])
        logit_parts.append(_gelu(x))
        hs_parts.append(hs_f)

    logit = jnp.concatenate(logit_parts, axis=1)
    lanes = jax.lax.broadcasted_iota(jnp.int32, (1, EP), 1)
    valid = lanes < E
    logit = jnp.where(valid, logit, np.float32(-1e30))

    iota_full = jax.lax.broadcasted_iota(jnp.int32, (N, EP), 0)
    bs = iota_full == src

    def _seg_sum(row):
        return jnp.sum(jnp.where(bs, row, 0.0), axis=1, keepdims=True)

    def _seg_gather(col):
        return jnp.sum(jnp.where(bs, jnp.broadcast_to(col, (N, EP)), 0.0),
                       axis=0, keepdims=True)

    zh = (logit + g) * np.float32(1.0 / TAU)
    zmask = jnp.where(bs, zh, np.float32(-1e30))
    m = jnp.max(zmask, axis=1, keepdims=True)
    eh = jnp.exp(zh - _seg_gather(m))
    eh = jnp.where(valid, eh, 0.0)
    hard = eh / (_seg_gather(_seg_sum(eh)) + np.float32(1e-12))
    es = jnp.exp(logit)
    es = jnp.where(valid, es, 0.0)
    soft = es / (_seg_gather(_seg_sum(es)) + np.float32(1e-12))
    coef = soft * hard
    coef = jnp.where(valid, coef, 0.0)

    acc = jnp.zeros((N, H), F32)
    for c in range(NCHUNK):
        sl = slice(c * TE, (c + 1) * TE)
        dst_c = dst[:, sl]
        iota_c = jax.lax.broadcasted_iota(jnp.int32, (N, TE), 0)
        sdw = _sp((iota_c == dst_c).astype(F32) * coef[:, sl])
        acc = acc + _dot3(sdw, _sp(hs_parts[c]), 1, 0)
    out_ref[0] = acc


def _splitw(w):
    wh = w.astype(BF)
    wl = (w - wh.astype(F32)).astype(BF)
    return jnp.stack([wh, wl])


def kernel(h, params, edge_index):
    f32 = jnp.float32
    p0, p1 = params['l0'], params['l1']
    a1 = p1['W_ih_r'][:, :H]
    a2 = p1['W_ih_r'][:, H:]
    f1 = p1['W_ih_f'][:, :H]
    f2 = p1['W_ih_f'][:, H:]

    wn = _splitw(jnp.concatenate([p0['W_ih_f'].T, p0['W_ih_r'].T], axis=1))
    bhf0 = jnp.stack([p0['b_ih_f'], p0['b_hh_f']], axis=0)
    bhr0 = jnp.stack([p0['b_ih_r'], p0['b_hh_r']], axis=0)

    w1 = _splitw(p0['W_ih_r'].T)
    b1 = p0['b_ih_r'][None, :]
    w2 = _splitw(jnp.concatenate([p0['W_hh_r'].T, a2.T], axis=1))
    b2 = jnp.concatenate([p0['b_hh_r'], jnp.zeros((G3,), f32)])[None, :]
    w3 = _splitw(p0['W_ih_f'].T)
    b3 = p0['b_ih_f'][None, :]
    w4 = _splitw(jnp.concatenate([p0['W_hh_f'].T, a1.T, f1.T], axis=1))
    b4 = jnp.concatenate([p0['b_hh_f'], jnp.zeros((2 * G3,), f32)])[None, :]
    w5 = _splitw(a1.T)
    w6 = _splitw(jnp.concatenate([a2.T, f2.T], axis=1))
    w7 = _splitw(p1['W_hh_r'].T)
    b7 = p1['b_hh_r'][None, :]
    bir1 = p1['b_ih_r'][None, :]
    bif1 = p1['b_ih_f'][None, :]
    bhr1 = p1['b_hh_r'][None, :]
    bhf1 = p1['b_hh_f'][None, :]
    wcf = _splitw(params['Wc'][0:1, :H])
    wcr = _splitw(params['Wc'][0:1, H:])
    bc = params['bc'][None, :]

    u = jax.random.uniform(jax.random.key(42), (E, B),
                           minval=1e-6, maxval=1.0 - 1e-6)
    g = -jnp.log(-jnp.log(u))
    gp = jnp.zeros((B, 1, EP), f32).at[:, 0, :E].set(jnp.transpose(g))

    pad = jnp.full((1, EP - E), N, jnp.int32)
    dstr = jnp.concatenate([edge_index[0][None, :], pad], axis=1)
    srcr = jnp.concatenate([edge_index[1][None, :], pad], axis=1)

    full = lambda shape: pl.BlockSpec(shape, lambda b: (0,) * len(shape))
    grid_spec = pl.GridSpec(
        grid=(B,),
        in_specs=[
            pl.BlockSpec((1, N, H), lambda b: (b, 0, 0)),
            full((1, EP)), full((1, EP)),
            pl.BlockSpec((1, 1, EP), lambda b: (b, 0, 0)),
            full((2, H, 2 * G3)), full((2, G3)), full((2, G3)),
            full((2, H, G3)), full((1, G3)),
            full((2, H, 2 * G3)), full((1, 2 * G3)),
            full((2, H, G3)), full((1, G3)),
            full((2, H, 3 * G3)), full((1, 3 * G3)),
            full((2, H, G3)),
            full((2, H, 2 * G3)),
            full((2, H, G3)), full((1, G3)),
            full((1, G3)), full((1, G3)), full((1, G3)), full((1, G3)),
            full((2, 1, H)), full((2, 1, H)), full((1, 1)),
        ],
        out_specs=pl.BlockSpec((1, N, H), lambda b: (b, 0, 0)),
    )
    out = pl.pallas_call(
        _body,
        grid_spec=grid_spec,
        out_shape=jax.ShapeDtypeStruct((B, N, H), f32),
    )(h, dstr, srcr, gp,
      wn, bhf0, bhr0,
      w1, b1, w2, b2, w3, b3, w4, b4,
      w5, w6, w7, b7,
      bir1, bif1, bhr1, bhf1,
      wcf, wcr, bc)
    return out


# speedup vs baseline: 3.5543x; 1.5338x over previous
import numpy as np
import jax
import jax.numpy as jnp
from jax.experimental import pallas as pl
from jax.experimental.pallas import tpu as pltpu

B, N, H, E = 16, 325, 64, 5200
TAU = 0.1
G3 = 3 * H
EP = 5376
TE = 896
NCHUNK = EP // TE
BF = jnp.bfloat16
F32 = jnp.float32


def _sp(a):
    ah = a.astype(BF)
    al = (a - ah.astype(F32)).astype(BF)
    return ah, al


def _dgb(a, b, ca, cb):
    return jax.lax.dot_general(
        a, b, (((ca,), (cb,)), ((), ())), preferred_element_type=F32)


def _dot3(asp, bsp, ca=1, cb=0):
    ah, al = asp
    bh, bl = bsp
    return (_dgb(ah, bh, ca, cb) + _dgb(ah, bl, ca, cb)
            + _dgb(al, bh, ca, cb))


def _dot2(mask_bf, bsp, ca, cb):
    bh, bl = bsp
    return _dgb(mask_bf, bh, ca, cb) + _dgb(mask_bf, bl, ca, cb)


def _gelu(x):
    return 0.5 * x * (1.0 + jax.lax.erf(x * np.float32(1.0 / np.sqrt(2.0))))


def _comb(gi, gh, hprev):
    r = jax.nn.sigmoid(gi[:, :H] + gh[:, :H])
    z = jax.nn.sigmoid(gi[:, H:2 * H] + gh[:, H:2 * H])
    n = jnp.tanh(gi[:, 2 * H:] + r * gh[:, 2 * H:])
    out = (1.0 - z) * n
    if hprev is not None:
        out = out + z * hprev
    return out


def _combP(po, hprev):
    r = jax.nn.sigmoid(po[:, :H])
    z = jax.nn.sigmoid(po[:, H:2 * H])
    n = jnp.tanh(po[:, 2 * H:3 * H] + r * po[:, 3 * H:])
    return (1.0 - z) * n + z * hprev


def _body(h_ref, dst_ref, src_ref, g_ref,
          wn_ref, bhf0_ref, bhr0_ref,
          wa1_ref, ba1_ref, wa2_ref, ba2_ref,
          wb1_ref, bb1_ref, wb2_ref, bb2_ref,
          wc1_ref, bc1_ref,
          bhr1_ref, bhf1_ref,
          wc_ref, bc_ref,
          out_ref):
    h = h_ref[0]
    dst = dst_ref[...]
    src = src_ref[...]
    g = g_ref[0]

    wsp = lambda ref: (ref[0], ref[1])

    hsp = _sp(h)
    gnode = _dot3(hsp, wsp(wn_ref))
    gf = gnode[:, :G3] + bhf0_ref[0:1]
    gr = gnode[:, G3:] + bhr0_ref[0:1]
    hf1 = _comb(gf, jnp.broadcast_to(bhf0_ref[1:2], (N, G3)), None)
    hr2 = _comb(gr, jnp.broadcast_to(bhr0_ref[1:2], (N, G3)), None)

    td = _sp(jnp.concatenate([h, hf1], axis=1))
    ts = _sp(jnp.concatenate([h, hr2], axis=1))

    logit_parts = []
    hs_parts = []
    for c in range(NCHUNK):
        sl = slice(c * TE, (c + 1) * TE)
        dst_c = dst[:, sl]
        src_c = src[:, sl]
        iota_c = jax.lax.broadcasted_iota(jnp.int32, (N, TE), 0)
        sd_c = (iota_c == dst_c).astype(BF)
        ss_c = (iota_c == src_c).astype(BF)
        gd = _dot2(sd_c, td, 0, 0)
        gs = _dot2(ss_c, ts, 0, 0)
        f1d = gd[:, H:]
        r2s = gs[:, H:]

        po1 = _dot3(_sp(jnp.concatenate([gd[:, :H], r2s], axis=1)),
                    wsp(wa1_ref)) + ba1_ref[...]
        hr1 = _combP(po1, r2s)
        po2 = _dot3(_sp(jnp.concatenate([gs[:, :H], f1d], axis=1)),
                    wsp(wa2_ref)) + ba2_ref[...]
        hf2 = _combP(po2, f1d)

        gi1 = _dot3(_sp(jnp.concatenate([hf2, r2s], axis=1)),
                    wsp(wb1_ref)) + bb1_ref[...]
        hr2l1 = _comb(gi1, jnp.broadcast_to(bhr1_ref[...], (TE, G3)), None)
        gif = _dot3(_sp(jnp.concatenate([f1d, hr1], axis=1)),
                    wsp(wb2_ref)) + bb2_ref[...]
        m7 = _dot3(_sp(hr2l1), wsp(wc1_ref)) + bc1_ref[...]
        o_r = _comb(gif[:, :G3], m7, hr2l1)
        o_f = _comb(gif[:, G3:], jnp.broadcast_to(bhf1_ref[...], (TE, G3)),
                    None)

        x = (_dot3(wsp(wc_ref),
                   _sp(jnp.concatenate([o_f, o_r], axis=1)), 1, 1)
             + bc_ref[0, 0])
        logit_parts.append(_gelu(x))
        hs_parts.append(gs[:, :H])

    logit = jnp.concatenate(logit_parts, axis=1)
    lanes = jax.lax.broadcasted_iota(jnp.int32, (1, EP), 1)
    valid = lanes < E
    logit = jnp.where(valid, logit, np.float32(-1e30))

    iota_full = jax.lax.broadcasted_iota(jnp.int32, (N, EP), 0)
    bs = iota_full == src

    def _seg_sum(row):
        return jnp.sum(jnp.where(bs, row, 0.0), axis=1, keepdims=True)

    def _seg_gather(col):
        return jnp.sum(jnp.where(bs, jnp.broadcast_to(col, (N, EP)), 0.0),
                       axis=0, keepdims=True)

    zh = (logit + g) * np.float32(1.0 / TAU)
    zmask = jnp.where(bs, zh, np.float32(-1e30))
    m = jnp.max(zmask, axis=1, keepdims=True)
    eh = jnp.exp(zh - _seg_gather(m))
    eh = jnp.where(valid, eh, 0.0)
    hard = eh / (_seg_gather(_seg_sum(eh)) + np.float32(1e-12))
    es = jnp.exp(logit)
    es = jnp.where(valid, es, 0.0)
    soft = es / (_seg_gather(_seg_sum(es)) + np.float32(1e-12))
    coef = soft * hard
    coef = jnp.where(valid, coef, 0.0)

    acc = jnp.zeros((N, H), F32)
    for c in range(NCHUNK):
        sl = slice(c * TE, (c + 1) * TE)
        dst_c = dst[:, sl]
        iota_c = jax.lax.broadcasted_iota(jnp.int32, (N, TE), 0)
        sdw = _sp((iota_c == dst_c).astype(F32) * coef[:, sl])
        acc = acc + _dot3(sdw, _sp(hs_parts[c]), 1, 0)
    out_ref[0] = acc


def _splitw(w):
    wh = w.astype(BF)
    wl = (w - wh.astype(F32)).astype(BF)
    return jnp.stack([wh, wl])


def kernel(h, params, edge_index):
    f32 = jnp.float32
    p0, p1 = params['l0'], params['l1']
    a1 = p1['W_ih_r'][:, :H]
    a2 = p1['W_ih_r'][:, H:]
    f1 = p1['W_ih_f'][:, :H]
    f2 = p1['W_ih_f'][:, H:]

    wn = _splitw(jnp.concatenate([p0['W_ih_f'].T, p0['W_ih_r'].T], axis=1))
    bhf0 = jnp.stack([p0['b_ih_f'], p0['b_hh_f']], axis=0)
    bhr0 = jnp.stack([p0['b_ih_r'], p0['b_hh_r']], axis=0)

    zH = jnp.zeros((H, H), f32)

    def _packA(wih, whh, bih, bhh):
        top = jnp.concatenate([wih.T[:, :2 * H], wih.T[:, 2 * H:], zH],
                              axis=1)
        bot = jnp.concatenate([whh.T[:, :2 * H], zH, whh.T[:, 2 * H:]],
                              axis=1)
        w = jnp.concatenate([top, bot], axis=0)
        b = jnp.concatenate([(bih + bhh)[:2 * H], bih[2 * H:], bhh[2 * H:]])
        return _splitw(w), b[None, :]

    wa1, ba1 = _packA(p0['W_ih_r'], p0['W_hh_r'],
                      p0['b_ih_r'], p0['b_hh_r'])
    wa2, ba2 = _packA(p0['W_ih_f'], p0['W_hh_f'],
                      p0['b_ih_f'], p0['b_hh_f'])
    wb1 = _splitw(jnp.concatenate([a1.T, a2.T], axis=0))
    bb1 = p1['b_ih_r'][None, :]
    wb2 = _splitw(jnp.concatenate(
        [jnp.concatenate([a1.T, a2.T], axis=0),
         jnp.concatenate([f1.T, f2.T], axis=0)], axis=1))
    bb2 = jnp.concatenate([p1['b_ih_r'], p1['b_ih_f']])[None, :]
    wc1 = _splitw(p1['W_hh_r'].T)
    bc1 = p1['b_hh_r'][None, :]
    bhr1 = p1['b_hh_r'][None, :]
    bhf1 = p1['b_hh_f'][None, :]
    wc = _splitw(params['Wc'][0:1, :])
    bc = params['bc'][None, :]

    u = jax.random.uniform(jax.random.key(42), (E, B),
                           minval=1e-6, maxval=1.0 - 1e-6)
    g = -jnp.log(-jnp.log(u))
    gp = jnp.zeros((B, 1, EP), f32).at[:, 0, :E].set(jnp.transpose(g))

    pad = jnp.full((1, EP - E), N, jnp.int32)
    dstr = jnp.concatenate([edge_index[0][None, :], pad], axis=1)
    srcr = jnp.concatenate([edge_index[1][None, :], pad], axis=1)

    full = lambda shape: pl.BlockSpec(shape, lambda b: (0,) * len(shape))
    grid_spec = pl.GridSpec(
        grid=(B,),
        in_specs=[
            pl.BlockSpec((1, N, H), lambda b: (b, 0, 0)),
            full((1, EP)), full((1, EP)),
            pl.BlockSpec((1, 1, EP), lambda b: (b, 0, 0)),
            full((2, H, 2 * G3)), full((2, G3)), full((2, G3)),
            full((2, 2 * H, 4 * H)), full((1, 4 * H)),
            full((2, 2 * H, 4 * H)), full((1, 4 * H)),
            full((2, 2 * H, G3)), full((1, G3)),
            full((2, 2 * H, 2 * G3)), full((1, 2 * G3)),
            full((2, H, G3)), full((1, G3)),
            full((1, G3)), full((1, G3)),
            full((2, 1, 2 * H)), full((1, 1)),
        ],
        out_specs=pl.BlockSpec((1, N, H), lambda b: (b, 0, 0)),
    )
    out = pl.pallas_call(
        _body,
        grid_spec=grid_spec,
        out_shape=jax.ShapeDtypeStruct((B, N, H), f32),
    )(h, dstr, srcr, gp,
      wn, bhf0, bhr0,
      wa1, ba1, wa2, ba2,
      wb1, bb1, wb2, bb2,
      wc1, bc1,
      bhr1, bhf1,
      wc, bc)
    return out


# speedup vs baseline: 3.7501x; 1.0551x over previous
import numpy as np
import jax
import jax.numpy as jnp
from jax.experimental import pallas as pl
from jax.experimental.pallas import tpu as pltpu

B, N, H, E = 16, 325, 64, 5200
TAU = 0.1
G3 = 3 * H
EP = 5376
TE = 1792
NCHUNK = EP // TE
BF = jnp.bfloat16
F32 = jnp.float32


def _sp(a):
    ah = a.astype(BF)
    al = (a - ah.astype(F32)).astype(BF)
    return ah, al


def _dgb(a, b, ca, cb):
    return jax.lax.dot_general(
        a, b, (((ca,), (cb,)), ((), ())), preferred_element_type=F32)


def _dot3(asp, bsp, ca=1, cb=0):
    ah, al = asp
    bh, bl = bsp
    return (_dgb(ah, bh, ca, cb) + _dgb(ah, bl, ca, cb)
            + _dgb(al, bh, ca, cb))


def _dot2(mask_bf, bsp, ca, cb):
    bh, bl = bsp
    return _dgb(mask_bf, bh, ca, cb) + _dgb(mask_bf, bl, ca, cb)


def _gelu(x):
    return 0.5 * x * (1.0 + jax.lax.erf(x * np.float32(1.0 / np.sqrt(2.0))))


def _comb(gi, gh, hprev):
    r = jax.nn.sigmoid(gi[:, :H] + gh[:, :H])
    z = jax.nn.sigmoid(gi[:, H:2 * H] + gh[:, H:2 * H])
    n = jnp.tanh(gi[:, 2 * H:] + r * gh[:, 2 * H:])
    out = (1.0 - z) * n
    if hprev is not None:
        out = out + z * hprev
    return out


def _combP(po, hprev):
    r = jax.nn.sigmoid(po[:, :H])
    z = jax.nn.sigmoid(po[:, H:2 * H])
    n = jnp.tanh(po[:, 2 * H:3 * H] + r * po[:, 3 * H:])
    return (1.0 - z) * n + z * hprev


def _body(h_ref, dst_ref, src_ref, g_ref,
          wn_ref, bhf0_ref, bhr0_ref,
          wa1_ref, ba1_ref, wa2_ref, ba2_ref,
          wb1_ref, bb1_ref, wb2_ref, bb2_ref,
          wc1_ref, bc1_ref,
          bhr1_ref, bhf1_ref,
          wc_ref, bc_ref,
          out_ref):
    h = h_ref[0]
    dst = dst_ref[...]
    src = src_ref[...]
    g = g_ref[0]

    wsp = lambda ref: (ref[0], ref[1])

    hsp = _sp(h)
    gnode = _dot3(hsp, wsp(wn_ref))
    gf = gnode[:, :G3] + bhf0_ref[0:1]
    gr = gnode[:, G3:] + bhr0_ref[0:1]
    hf1 = _comb(gf, jnp.broadcast_to(bhf0_ref[1:2], (N, G3)), None)
    hr2 = _comb(gr, jnp.broadcast_to(bhr0_ref[1:2], (N, G3)), None)

    td = _sp(jnp.concatenate([h, hf1], axis=1))
    ts = _sp(jnp.concatenate([h, hr2], axis=1))

    logit_parts = []
    hs_parts = []
    for c in range(NCHUNK):
        sl = slice(c * TE, (c + 1) * TE)
        dst_c = dst[:, sl]
        src_c = src[:, sl]
        iota_c = jax.lax.broadcasted_iota(jnp.int32, (N, TE), 0)
        sd_c = (iota_c == dst_c).astype(BF)
        ss_c = (iota_c == src_c).astype(BF)
        gd = _dot2(sd_c, td, 0, 0)
        gs = _dot2(ss_c, ts, 0, 0)
        f1d = gd[:, H:]
        r2s = gs[:, H:]

        po1 = _dot3(_sp(jnp.concatenate([gd[:, :H], r2s], axis=1)),
                    wsp(wa1_ref)) + ba1_ref[...]
        hr1 = _combP(po1, r2s)
        po2 = _dot3(_sp(jnp.concatenate([gs[:, :H], f1d], axis=1)),
                    wsp(wa2_ref)) + ba2_ref[...]
        hf2 = _combP(po2, f1d)

        gi1 = _dot3(_sp(jnp.concatenate([hf2, r2s], axis=1)),
                    wsp(wb1_ref)) + bb1_ref[...]
        hr2l1 = _comb(gi1, jnp.broadcast_to(bhr1_ref[...], (TE, G3)), None)
        gif = _dot3(_sp(jnp.concatenate([f1d, hr1], axis=1)),
                    wsp(wb2_ref)) + bb2_ref[...]
        m7 = _dot3(_sp(hr2l1), wsp(wc1_ref)) + bc1_ref[...]
        o_r = _comb(gif[:, :G3], m7, hr2l1)
        o_f = _comb(gif[:, G3:], jnp.broadcast_to(bhf1_ref[...], (TE, G3)),
                    None)

        x = (_dot3(wsp(wc_ref),
                   _sp(jnp.concatenate([o_f, o_r], axis=1)), 1, 1)
             + bc_ref[0, 0])
        logit_parts.append(_gelu(x))
        hs_parts.append(gs[:, :H])

    logit = jnp.concatenate(logit_parts, axis=1)
    lanes = jax.lax.broadcasted_iota(jnp.int32, (1, EP), 1)
    valid = lanes < E
    logit = jnp.where(valid, logit, np.float32(-1e30))

    iota_full = jax.lax.broadcasted_iota(jnp.int32, (N, EP), 0)
    bs = iota_full == src

    def _seg_sum(row):
        return jnp.sum(jnp.where(bs, row, 0.0), axis=1, keepdims=True)

    def _seg_gather(col):
        return jnp.sum(jnp.where(bs, jnp.broadcast_to(col, (N, EP)), 0.0),
                       axis=0, keepdims=True)

    zh = (logit + g) * np.float32(1.0 / TAU)
    zmask = jnp.where(bs, zh, np.float32(-1e30))
    m = jnp.max(zmask, axis=1, keepdims=True)
    eh = jnp.exp(zh - _seg_gather(m))
    eh = jnp.where(valid, eh, 0.0)
    hard = eh / (_seg_gather(_seg_sum(eh)) + np.float32(1e-12))
    es = jnp.exp(logit)
    es = jnp.where(valid, es, 0.0)
    soft = es / (_seg_gather(_seg_sum(es)) + np.float32(1e-12))
    coef = soft * hard
    coef = jnp.where(valid, coef, 0.0)

    acc = jnp.zeros((N, H), F32)
    for c in range(NCHUNK):
        sl = slice(c * TE, (c + 1) * TE)
        dst_c = dst[:, sl]
        iota_c = jax.lax.broadcasted_iota(jnp.int32, (N, TE), 0)
        sdw = _sp((iota_c == dst_c).astype(F32) * coef[:, sl])
        acc = acc + _dot3(sdw, _sp(hs_parts[c]), 1, 0)
    out_ref[0] = acc


def _splitw(w):
    wh = w.astype(BF)
    wl = (w - wh.astype(F32)).astype(BF)
    return jnp.stack([wh, wl])


def kernel(h, params, edge_index):
    f32 = jnp.float32
    p0, p1 = params['l0'], params['l1']
    a1 = p1['W_ih_r'][:, :H]
    a2 = p1['W_ih_r'][:, H:]
    f1 = p1['W_ih_f'][:, :H]
    f2 = p1['W_ih_f'][:, H:]

    wn = _splitw(jnp.concatenate([p0['W_ih_f'].T, p0['W_ih_r'].T], axis=1))
    bhf0 = jnp.stack([p0['b_ih_f'], p0['b_hh_f']], axis=0)
    bhr0 = jnp.stack([p0['b_ih_r'], p0['b_hh_r']], axis=0)

    zH = jnp.zeros((H, H), f32)

    def _packA(wih, whh, bih, bhh):
        top = jnp.concatenate([wih.T[:, :2 * H], wih.T[:, 2 * H:], zH],
                              axis=1)
        bot = jnp.concatenate([whh.T[:, :2 * H], zH, whh.T[:, 2 * H:]],
                              axis=1)
        w = jnp.concatenate([top, bot], axis=0)
        b = jnp.concatenate([(bih + bhh)[:2 * H], bih[2 * H:], bhh[2 * H:]])
        return _splitw(w), b[None, :]

    wa1, ba1 = _packA(p0['W_ih_r'], p0['W_hh_r'],
                      p0['b_ih_r'], p0['b_hh_r'])
    wa2, ba2 = _packA(p0['W_ih_f'], p0['W_hh_f'],
                      p0['b_ih_f'], p0['b_hh_f'])
    wb1 = _splitw(jnp.concatenate([a1.T, a2.T], axis=0))
    bb1 = p1['b_ih_r'][None, :]
    wb2 = _splitw(jnp.concatenate(
        [jnp.concatenate([a1.T, a2.T], axis=0),
         jnp.concatenate([f1.T, f2.T], axis=0)], axis=1))
    bb2 = jnp.concatenate([p1['b_ih_r'], p1['b_ih_f']])[None, :]
    wc1 = _splitw(p1['W_hh_r'].T)
    bc1 = p1['b_hh_r'][None, :]
    bhr1 = p1['b_hh_r'][None, :]
    bhf1 = p1['b_hh_f'][None, :]
    wc = _splitw(params['Wc'][0:1, :])
    bc = params['bc'][None, :]

    u = jax.random.uniform(jax.random.key(42), (E, B),
                           minval=1e-6, maxval=1.0 - 1e-6)
    g = -jnp.log(-jnp.log(u))
    gp = jnp.zeros((B, 1, EP), f32).at[:, 0, :E].set(jnp.transpose(g))

    pad = jnp.full((1, EP - E), N, jnp.int32)
    dstr = jnp.concatenate([edge_index[0][None, :], pad], axis=1)
    srcr = jnp.concatenate([edge_index[1][None, :], pad], axis=1)

    full = lambda shape: pl.BlockSpec(shape, lambda b: (0,) * len(shape))
    grid_spec = pl.GridSpec(
        grid=(B,),
        in_specs=[
            pl.BlockSpec((1, N, H), lambda b: (b, 0, 0)),
            full((1, EP)), full((1, EP)),
            pl.BlockSpec((1, 1, EP), lambda b: (b, 0, 0)),
            full((2, H, 2 * G3)), full((2, G3)), full((2, G3)),
            full((2, 2 * H, 4 * H)), full((1, 4 * H)),
            full((2, 2 * H, 4 * H)), full((1, 4 * H)),
            full((2, 2 * H, G3)), full((1, G3)),
            full((2, 2 * H, 2 * G3)), full((1, 2 * G3)),
            full((2, H, G3)), full((1, G3)),
            full((1, G3)), full((1, G3)),
            full((2, 1, 2 * H)), full((1, 1)),
        ],
        out_specs=pl.BlockSpec((1, N, H), lambda b: (b, 0, 0)),
    )
    out = pl.pallas_call(
        _body,
        grid_spec=grid_spec,
        out_shape=jax.ShapeDtypeStruct((B, N, H), f32),
    )(h, dstr, srcr, gp,
      wn, bhf0, bhr0,
      wa1, ba1, wa2, ba2,
      wb1, bb1, wb2, bb2,
      wc1, bc1,
      bhr1, bhf1,
      wc, bc)
    return out


# speedup vs baseline: 3.7524x; 1.0006x over previous
import numpy as np
import jax
import jax.numpy as jnp
from jax.experimental import pallas as pl
from jax.experimental.pallas import tpu as pltpu

B, N, H, E = 16, 325, 64, 5200
TAU = 0.1
G3 = 3 * H
EP = 5376
TE = 2688
NCHUNK = EP // TE
BF = jnp.bfloat16
F32 = jnp.float32


def _sp(a):
    ah = a.astype(BF)
    al = (a - ah.astype(F32)).astype(BF)
    return ah, al


def _dgb(a, b, ca, cb):
    return jax.lax.dot_general(
        a, b, (((ca,), (cb,)), ((), ())), preferred_element_type=F32)


def _dot3(asp, bsp, ca=1, cb=0):
    ah, al = asp
    bh, bl = bsp
    return (_dgb(ah, bh, ca, cb) + _dgb(ah, bl, ca, cb)
            + _dgb(al, bh, ca, cb))


def _dot2(mask_bf, bsp, ca, cb):
    bh, bl = bsp
    return _dgb(mask_bf, bh, ca, cb) + _dgb(mask_bf, bl, ca, cb)


def _gelu(x):
    return 0.5 * x * (1.0 + jax.lax.erf(x * np.float32(1.0 / np.sqrt(2.0))))


def _comb(gi, gh, hprev):
    r = jax.nn.sigmoid(gi[:, :H] + gh[:, :H])
    z = jax.nn.sigmoid(gi[:, H:2 * H] + gh[:, H:2 * H])
    n = jnp.tanh(gi[:, 2 * H:] + r * gh[:, 2 * H:])
    out = (1.0 - z) * n
    if hprev is not None:
        out = out + z * hprev
    return out


def _combP(po, hprev):
    r = jax.nn.sigmoid(po[:, :H])
    z = jax.nn.sigmoid(po[:, H:2 * H])
    n = jnp.tanh(po[:, 2 * H:3 * H] + r * po[:, 3 * H:])
    return (1.0 - z) * n + z * hprev


def _body(h_ref, dst_ref, src_ref, g_ref,
          wn_ref, bhf0_ref, bhr0_ref,
          wa1_ref, ba1_ref, wa2_ref, ba2_ref,
          wb1_ref, bb1_ref, wb2_ref, bb2_ref,
          wc1_ref, bc1_ref,
          bhr1_ref, bhf1_ref,
          wc_ref, bc_ref,
          out_ref):
    h = h_ref[0]
    dst = dst_ref[...]
    src = src_ref[...]
    g = g_ref[0]

    wsp = lambda ref: (ref[0], ref[1])

    hsp = _sp(h)
    gnode = _dot3(hsp, wsp(wn_ref))
    gf = gnode[:, :G3] + bhf0_ref[0:1]
    gr = gnode[:, G3:] + bhr0_ref[0:1]
    hf1 = _comb(gf, jnp.broadcast_to(bhf0_ref[1:2], (N, G3)), None)
    hr2 = _comb(gr, jnp.broadcast_to(bhr0_ref[1:2], (N, G3)), None)

    td = _sp(jnp.concatenate([h, hf1], axis=1))
    ts = _sp(jnp.concatenate([h, hr2], axis=1))

    logit_parts = []
    hs_parts = []
    for c in range(NCHUNK):
        sl = slice(c * TE, (c + 1) * TE)
        dst_c = dst[:, sl]
        src_c = src[:, sl]
        iota_c = jax.lax.broadcasted_iota(jnp.int32, (N, TE), 0)
        sd_c = (iota_c == dst_c).astype(BF)
        ss_c = (iota_c == src_c).astype(BF)
        gd = _dot2(sd_c, td, 0, 0)
        gs = _dot2(ss_c, ts, 0, 0)
        f1d = gd[:, H:]
        r2s = gs[:, H:]

        po1 = _dot3(_sp(jnp.concatenate([gd[:, :H], r2s], axis=1)),
                    wsp(wa1_ref)) + ba1_ref[...]
        hr1 = _combP(po1, r2s)
        po2 = _dot3(_sp(jnp.concatenate([gs[:, :H], f1d], axis=1)),
                    wsp(wa2_ref)) + ba2_ref[...]
        hf2 = _combP(po2, f1d)

        gi1 = _dot3(_sp(jnp.concatenate([hf2, r2s], axis=1)),
                    wsp(wb1_ref)) + bb1_ref[...]
        hr2l1 = _comb(gi1, jnp.broadcast_to(bhr1_ref[...], (TE, G3)), None)
        gif = _dot3(_sp(jnp.concatenate([f1d, hr1], axis=1)),
                    wsp(wb2_ref)) + bb2_ref[...]
        m7 = _dot3(_sp(hr2l1), wsp(wc1_ref)) + bc1_ref[...]
        o_r = _comb(gif[:, :G3], m7, hr2l1)
        o_f = _comb(gif[:, G3:], jnp.broadcast_to(bhf1_ref[...], (TE, G3)),
                    None)

        x = (_dot3(wsp(wc_ref),
                   _sp(jnp.concatenate([o_f, o_r], axis=1)), 1, 1)
             + bc_ref[0, 0])
        logit_parts.append(_gelu(x))
        hs_parts.append(gs[:, :H])

    logit = jnp.concatenate(logit_parts, axis=1)
    lanes = jax.lax.broadcasted_iota(jnp.int32, (1, EP), 1)
    valid = lanes < E
    logit = jnp.where(valid, logit, np.float32(-1e30))

    iota_full = jax.lax.broadcasted_iota(jnp.int32, (N, EP), 0)
    bs = iota_full == src

    def _seg_sum(row):
        return jnp.sum(jnp.where(bs, row, 0.0), axis=1, keepdims=True)

    def _seg_gather(col):
        return jnp.sum(jnp.where(bs, jnp.broadcast_to(col, (N, EP)), 0.0),
                       axis=0, keepdims=True)

    zh = (logit + g) * np.float32(1.0 / TAU)
    zmask = jnp.where(bs, zh, np.float32(-1e30))
    m = jnp.max(zmask, axis=1, keepdims=True)
    eh = jnp.exp(zh - _seg_gather(m))
    eh = jnp.where(valid, eh, 0.0)
    hard = eh / (_seg_gather(_seg_sum(eh)) + np.float32(1e-12))
    es = jnp.exp(logit)
    es = jnp.where(valid, es, 0.0)
    soft = es / (_seg_gather(_seg_sum(es)) + np.float32(1e-12))
    coef = soft * hard
    coef = jnp.where(valid, coef, 0.0)

    acc = jnp.zeros((N, H), F32)
    for c in range(NCHUNK):
        sl = slice(c * TE, (c + 1) * TE)
        dst_c = dst[:, sl]
        iota_c = jax.lax.broadcasted_iota(jnp.int32, (N, TE), 0)
        sdw = _sp((iota_c == dst_c).astype(F32) * coef[:, sl])
        acc = acc + _dot3(sdw, _sp(hs_parts[c]), 1, 0)
    out_ref[0] = acc


def _splitw(w):
    wh = w.astype(BF)
    wl = (w - wh.astype(F32)).astype(BF)
    return jnp.stack([wh, wl])


def kernel(h, params, edge_index):
    f32 = jnp.float32
    p0, p1 = params['l0'], params['l1']
    a1 = p1['W_ih_r'][:, :H]
    a2 = p1['W_ih_r'][:, H:]
    f1 = p1['W_ih_f'][:, :H]
    f2 = p1['W_ih_f'][:, H:]

    wn = _splitw(jnp.concatenate([p0['W_ih_f'].T, p0['W_ih_r'].T], axis=1))
    bhf0 = jnp.stack([p0['b_ih_f'], p0['b_hh_f']], axis=0)
    bhr0 = jnp.stack([p0['b_ih_r'], p0['b_hh_r']], axis=0)

    zH = jnp.zeros((H, H), f32)

    def _packA(wih, whh, bih, bhh):
        top = jnp.concatenate([wih.T[:, :2 * H], wih.T[:, 2 * H:], zH],
                              axis=1)
        bot = jnp.concatenate([whh.T[:, :2 * H], zH, whh.T[:, 2 * H:]],
                              axis=1)
        w = jnp.concatenate([top, bot], axis=0)
        b = jnp.concatenate([(bih + bhh)[:2 * H], bih[2 * H:], bhh[2 * H:]])
        return _splitw(w), b[None, :]

    wa1, ba1 = _packA(p0['W_ih_r'], p0['W_hh_r'],
                      p0['b_ih_r'], p0['b_hh_r'])
    wa2, ba2 = _packA(p0['W_ih_f'], p0['W_hh_f'],
                      p0['b_ih_f'], p0['b_hh_f'])
    wb1 = _splitw(jnp.concatenate([a1.T, a2.T], axis=0))
    bb1 = p1['b_ih_r'][None, :]
    wb2 = _splitw(jnp.concatenate(
        [jnp.concatenate([a1.T, a2.T], axis=0),
         jnp.concatenate([f1.T, f2.T], axis=0)], axis=1))
    bb2 = jnp.concatenate([p1['b_ih_r'], p1['b_ih_f']])[None, :]
    wc1 = _splitw(p1['W_hh_r'].T)
    bc1 = p1['b_hh_r'][None, :]
    bhr1 = p1['b_hh_r'][None, :]
    bhf1 = p1['b_hh_f'][None, :]
    wc = _splitw(params['Wc'][0:1, :])
    bc = params['bc'][None, :]

    u = jax.random.uniform(jax.random.key(42), (E, B),
                           minval=1e-6, maxval=1.0 - 1e-6)
    g = -jnp.log(-jnp.log(u))
    gp = jnp.zeros((B, 1, EP), f32).at[:, 0, :E].set(jnp.transpose(g))

    pad = jnp.full((1, EP - E), N, jnp.int32)
    dstr = jnp.concatenate([edge_index[0][None, :], pad], axis=1)
    srcr = jnp.concatenate([edge_index[1][None, :], pad], axis=1)

    full = lambda shape: pl.BlockSpec(shape, lambda b: (0,) * len(shape))
    grid_spec = pl.GridSpec(
        grid=(B,),
        in_specs=[
            pl.BlockSpec((1, N, H), lambda b: (b, 0, 0)),
            full((1, EP)), full((1, EP)),
            pl.BlockSpec((1, 1, EP), lambda b: (b, 0, 0)),
            full((2, H, 2 * G3)), full((2, G3)), full((2, G3)),
            full((2, 2 * H, 4 * H)), full((1, 4 * H)),
            full((2, 2 * H, 4 * H)), full((1, 4 * H)),
            full((2, 2 * H, G3)), full((1, G3)),
            full((2, 2 * H, 2 * G3)), full((1, 2 * G3)),
            full((2, H, G3)), full((1, G3)),
            full((1, G3)), full((1, G3)),
            full((2, 1, 2 * H)), full((1, 1)),
        ],
        out_specs=pl.BlockSpec((1, N, H), lambda b: (b, 0, 0)),
    )
    out = pl.pallas_call(
        _body,
        grid_spec=grid_spec,
        out_shape=jax.ShapeDtypeStruct((B, N, H), f32),
    )(h, dstr, srcr, gp,
      wn, bhf0, bhr0,
      wa1, ba1, wa2, ba2,
      wb1, bb1, wb2, bb2,
      wc1, bc1,
      bhr1, bhf1,
      wc, bc)
    return out


# speedup vs baseline: 4.2287x; 1.1269x over previous
import numpy as np
import jax
import jax.numpy as jnp
from jax.experimental import pallas as pl
from jax.experimental.pallas import tpu as pltpu

B, N, H, E = 16, 325, 64, 5200
TAU = 0.1
G3 = 3 * H
EP = 5376
TE = 896
NCHUNK = EP // TE
KB = 4
NP = 328
BF = jnp.bfloat16
F32 = jnp.float32


def _sp(a):
    ah = a.astype(BF)
    al = (a - ah.astype(F32)).astype(BF)
    return ah, al


def _dgb(a, b, ca, cb):
    return jax.lax.dot_general(
        a, b, (((ca,), (cb,)), ((), ())), preferred_element_type=F32)


def _dot3(asp, bsp, ca=1, cb=0):
    ah, al = asp
    bh, bl = bsp
    return (_dgb(ah, bh, ca, cb) + _dgb(ah, bl, ca, cb)
            + _dgb(al, bh, ca, cb))


def _dot2(mask_bf, bsp, ca, cb):
    bh, bl = bsp
    return _dgb(mask_bf, bh, ca, cb) + _dgb(mask_bf, bl, ca, cb)


def _gelu(x):
    return 0.5 * x * (1.0 + jax.lax.erf(x * np.float32(1.0 / np.sqrt(2.0))))


def _comb(gi, gh, hprev):
    r = jax.nn.sigmoid(gi[:, :H] + gh[:, :H])
    z = jax.nn.sigmoid(gi[:, H:2 * H] + gh[:, H:2 * H])
    n = jnp.tanh(gi[:, 2 * H:] + r * gh[:, 2 * H:])
    out = (1.0 - z) * n
    if hprev is not None:
        out = out + z * hprev
    return out


def _combP(po, hprev):
    r = jax.nn.sigmoid(po[:, :H])
    z = jax.nn.sigmoid(po[:, H:2 * H])
    n = jnp.tanh(po[:, 2 * H:3 * H] + r * po[:, 3 * H:])
    return (1.0 - z) * n + z * hprev


def _body(h_ref, dst_ref, src_ref, g_ref,
          wn_ref, bhf0_ref, bhr0_ref,
          wa1_ref, ba1_ref, wa2_ref, ba2_ref,
          wb1_ref, bb1_ref, wb2_ref, bb2_ref,
          wc1_ref, bc1_ref,
          bhr1_ref, bhf1_ref,
          wc_ref, bc_ref,
          out_ref):
    hall = h_ref[...].reshape(KB * NP, H)
    dst = dst_ref[...]
    src = src_ref[...]
    g = g_ref[...].reshape(KB, EP)

    wsp = lambda ref: (ref[0], ref[1])

    hsp = _sp(hall)
    gnode = _dot3(hsp, wsp(wn_ref))
    gf = gnode[:, :G3] + bhf0_ref[0:1]
    gr = gnode[:, G3:] + bhr0_ref[0:1]
    hf1 = _comb(gf, jnp.broadcast_to(bhf0_ref[1:2], (KB * NP, G3)), None)
    hr2 = _comb(gr, jnp.broadcast_to(bhr0_ref[1:2], (KB * NP, G3)), None)

    td = _sp(jnp.concatenate(
        [jnp.concatenate([hall[b * NP:(b + 1) * NP], hf1[b * NP:(b + 1) * NP]],
                         axis=1) for b in range(KB)], axis=1))
    ts = _sp(jnp.concatenate(
        [jnp.concatenate([hr2[b * NP:(b + 1) * NP], hall[b * NP:(b + 1) * NP]],
                         axis=1) for b in range(KB)], axis=1))

    logit_parts = []
    hs_parts = []
    sd_masks = []
    R = KB * TE
    for c in range(NCHUNK):
        sl = slice(c * TE, (c + 1) * TE)
        dst_c = dst[:, sl]
        src_c = src[:, sl]
        iota_c = jax.lax.broadcasted_iota(jnp.int32, (NP, TE), 0)
        sd_c = (iota_c == dst_c).astype(BF)
        ss_c = (iota_c == src_c).astype(BF)
        sd_masks.append(sd_c)
        gd = _dot2(sd_c, td, 0, 0)
        gs = _dot2(ss_c, ts, 0, 0)
        hd = jnp.concatenate([gd[:, b * 2 * H:b * 2 * H + H]
                              for b in range(KB)], axis=0)
        f1d = jnp.concatenate([gd[:, b * 2 * H + H:(b + 1) * 2 * H]
                               for b in range(KB)], axis=0)
        r2s = jnp.concatenate([gs[:, b * 2 * H:b * 2 * H + H]
                               for b in range(KB)], axis=0)
        hs = jnp.concatenate([gs[:, b * 2 * H + H:(b + 1) * 2 * H]
                              for b in range(KB)], axis=0)

        po1 = _dot3(_sp(jnp.concatenate([hd, r2s], axis=1)),
                    wsp(wa1_ref)) + ba1_ref[...]
        hr1 = _combP(po1, r2s)
        po2 = _dot3(_sp(jnp.concatenate([hs, f1d], axis=1)),
                    wsp(wa2_ref)) + ba2_ref[...]
        hf2 = _combP(po2, f1d)

        gi1 = _dot3(_sp(jnp.concatenate([hf2, r2s], axis=1)),
                    wsp(wb1_ref)) + bb1_ref[...]
        hr2l1 = _comb(gi1, jnp.broadcast_to(bhr1_ref[...], (R, G3)), None)
        gif = _dot3(_sp(jnp.concatenate([f1d, hr1], axis=1)),
                    wsp(wb2_ref)) + bb2_ref[...]
        m7 = _dot3(_sp(hr2l1), wsp(wc1_ref)) + bc1_ref[...]
        o_r = _comb(gif[:, :G3], m7, hr2l1)
        o_f = _comb(gif[:, G3:], jnp.broadcast_to(bhf1_ref[...], (R, G3)),
                    None)

        x = (_dot3(wsp(wc_ref),
                   _sp(jnp.concatenate([o_f, o_r], axis=1)), 1, 1)
             + bc_ref[0, 0])
        logit_parts.append(_gelu(x).reshape(KB, TE))
        hs_parts.append(hs)

    logit = jnp.concatenate(logit_parts, axis=1)
    lanes = jax.lax.broadcasted_iota(jnp.int32, (1, EP), 1)
    valid = lanes < E
    logit = jnp.where(valid, logit, np.float32(-1e30))

    iota_full = jax.lax.broadcasted_iota(jnp.int32, (NP, EP), 0)
    bs = iota_full == src

    def _seg_sum(row):
        return jnp.sum(jnp.where(bs, row, 0.0), axis=1, keepdims=True)

    def _seg_gather(col):
        return jnp.sum(jnp.where(bs, jnp.broadcast_to(col, (NP, EP)), 0.0),
                       axis=0, keepdims=True)

    zh = (logit + g) * np.float32(1.0 / TAU)
    es_all = jnp.where(valid, jnp.exp(logit), 0.0)
    coef_rows = []
    for b in range(KB):
        zb = zh[b:b + 1]
        m = jnp.max(jnp.where(bs, zb, np.float32(-1e30)), axis=1,
                    keepdims=True)
        eh = jnp.exp(zb - _seg_gather(m))
        eh = jnp.where(valid, eh, 0.0)
        hard = eh / (_seg_gather(_seg_sum(eh)) + np.float32(1e-12))
        es = es_all[b:b + 1]
        soft = es / (_seg_gather(_seg_sum(es)) + np.float32(1e-12))
        coef_rows.append(jnp.where(valid, soft * hard, 0.0))

    for b in range(KB):
        coef = coef_rows[b]
        acc = jnp.zeros((NP, H), F32)
        for c in range(NCHUNK):
            sl = slice(c * TE, (c + 1) * TE)
            sdw = _sp(sd_masks[c].astype(F32) * coef[:, sl])
            hs_b = hs_parts[c][b * TE:(b + 1) * TE]
            acc = acc + _dot3(sdw, _sp(hs_b), 1, 0)
        out_ref[b] = acc


def _splitw(w):
    wh = w.astype(BF)
    wl = (w - wh.astype(F32)).astype(BF)
    return jnp.stack([wh, wl])


def kernel(h, params, edge_index):
    f32 = jnp.float32
    p0, p1 = params['l0'], params['l1']
    a1 = p1['W_ih_r'][:, :H]
    a2 = p1['W_ih_r'][:, H:]
    f1 = p1['W_ih_f'][:, :H]
    f2 = p1['W_ih_f'][:, H:]

    wn = _splitw(jnp.concatenate([p0['W_ih_f'].T, p0['W_ih_r'].T], axis=1))
    bhf0 = jnp.stack([p0['b_ih_f'], p0['b_hh_f']], axis=0)
    bhr0 = jnp.stack([p0['b_ih_r'], p0['b_hh_r']], axis=0)

    zH = jnp.zeros((H, H), f32)

    def _packA(wih, whh, bih, bhh):
        top = jnp.concatenate([wih.T[:, :2 * H], wih.T[:, 2 * H:], zH],
                              axis=1)
        bot = jnp.concatenate([whh.T[:, :2 * H], zH, whh.T[:, 2 * H:]],
                              axis=1)
        w = jnp.concatenate([top, bot], axis=0)
        b = jnp.concatenate([(bih + bhh)[:2 * H], bih[2 * H:], bhh[2 * H:]])
        return _splitw(w), b[None, :]

    wa1, ba1 = _packA(p0['W_ih_r'], p0['W_hh_r'],
                      p0['b_ih_r'], p0['b_hh_r'])
    wa2, ba2 = _packA(p0['W_ih_f'], p0['W_hh_f'],
                      p0['b_ih_f'], p0['b_hh_f'])
    wb1 = _splitw(jnp.concatenate([a1.T, a2.T], axis=0))
    bb1 = p1['b_ih_r'][None, :]
    wb2 = _splitw(jnp.concatenate(
        [jnp.concatenate([a1.T, a2.T], axis=0),
         jnp.concatenate([f1.T, f2.T], axis=0)], axis=1))
    bb2 = jnp.concatenate([p1['b_ih_r'], p1['b_ih_f']])[None, :]
    wc1 = _splitw(p1['W_hh_r'].T)
    bc1 = p1['b_hh_r'][None, :]
    bhr1 = p1['b_hh_r'][None, :]
    bhf1 = p1['b_hh_f'][None, :]
    wc = _splitw(params['Wc'][0:1, :])
    bc = params['bc'][None, :]

    u = jax.random.uniform(jax.random.key(42), (E, B),
                           minval=1e-6, maxval=1.0 - 1e-6)
    g = -jnp.log(-jnp.log(u))
    gp = jnp.zeros((B, 1, EP), f32).at[:, 0, :E].set(jnp.transpose(g))

    pad = jnp.full((1, EP - E), N, jnp.int32)
    dstr = jnp.concatenate([edge_index[0][None, :], pad], axis=1)
    srcr = jnp.concatenate([edge_index[1][None, :], pad], axis=1)

    hp = jnp.zeros((B, NP, H), f32).at[:, :N, :].set(h)

    full = lambda shape: pl.BlockSpec(shape, lambda b: (0,) * len(shape))
    grid_spec = pl.GridSpec(
        grid=(B // KB,),
        in_specs=[
            pl.BlockSpec((KB, NP, H), lambda b: (b, 0, 0)),
            full((1, EP)), full((1, EP)),
            pl.BlockSpec((KB, 1, EP), lambda b: (b, 0, 0)),
            full((2, H, 2 * G3)), full((2, G3)), full((2, G3)),
            full((2, 2 * H, 4 * H)), full((1, 4 * H)),
            full((2, 2 * H, 4 * H)), full((1, 4 * H)),
            full((2, 2 * H, G3)), full((1, G3)),
            full((2, 2 * H, 2 * G3)), full((1, 2 * G3)),
            full((2, H, G3)), full((1, G3)),
            full((1, G3)), full((1, G3)),
            full((2, 1, 2 * H)), full((1, 1)),
        ],
        out_specs=pl.BlockSpec((KB, NP, H), lambda b: (b, 0, 0)),
    )
    out = pl.pallas_call(
        _body,
        grid_spec=grid_spec,
        out_shape=jax.ShapeDtypeStruct((B, NP, H), f32),
    )(hp, dstr, srcr, gp,
      wn, bhf0, bhr0,
      wa1, ba1, wa2, ba2,
      wb1, bb1, wb2, bb2,
      wc1, bc1,
      bhr1, bhf1,
      wc, bc)
    return out[:, :N, :]


# speedup vs baseline: 4.6893x; 1.1089x over previous
import numpy as np
import jax
import jax.numpy as jnp
from jax.experimental import pallas as pl
from jax.experimental.pallas import tpu as pltpu

B, N, H, E = 16, 325, 64, 5200
TAU = 0.1
G3 = 3 * H
EP = 5376
TE = 896
NCHUNK = EP // TE
KB = 4
NP = 328
BF = jnp.bfloat16
F32 = jnp.float32


def _sp(a):
    ah = a.astype(BF)
    al = (a - ah.astype(F32)).astype(BF)
    return ah, al


def _dgb(a, b, ca, cb):
    return jax.lax.dot_general(
        a, b, (((ca,), (cb,)), ((), ())), preferred_element_type=F32)


def _dot3(asp, bsp, ca=1, cb=0):
    ah, al = asp
    bh, bl = bsp
    return (_dgb(ah, bh, ca, cb) + _dgb(ah, bl, ca, cb)
            + _dgb(al, bh, ca, cb))


def _dot2(mask_bf, bsp, ca, cb):
    bh, bl = bsp
    return _dgb(mask_bf, bh, ca, cb) + _dgb(mask_bf, bl, ca, cb)


def _gelu(x):
    return 0.5 * x * (1.0 + jax.lax.erf(x * np.float32(1.0 / np.sqrt(2.0))))


def _comb(gi, gh, hprev):
    r = jax.nn.sigmoid(gi[:, :H] + gh[:, :H])
    z = jax.nn.sigmoid(gi[:, H:2 * H] + gh[:, H:2 * H])
    n = jnp.tanh(gi[:, 2 * H:] + r * gh[:, 2 * H:])
    out = (1.0 - z) * n
    if hprev is not None:
        out = out + z * hprev
    return out


def _combP(po, hprev):
    r = jax.nn.sigmoid(po[:, :H])
    z = jax.nn.sigmoid(po[:, H:2 * H])
    n = jnp.tanh(po[:, 2 * H:3 * H] + r * po[:, 3 * H:])
    return (1.0 - z) * n + z * hprev


def _body(h_ref, dst_ref, src_ref, g_ref,
          wn_ref, bhf0_ref, bhr0_ref,
          wa1_ref, ba1_ref, wa2_ref, ba2_ref,
          wb1_ref, bb1_ref, wb2_ref, bb2_ref,
          wc1_ref, bc1_ref,
          bhr1_ref, bhf1_ref,
          wc_ref, bc_ref,
          out_ref):
    hall = h_ref[...].reshape(KB * NP, H)
    dst = dst_ref[...]
    src = src_ref[...]
    g = g_ref[...].reshape(KB, EP)

    wsp = lambda ref: (ref[0], ref[1])

    hsp = _sp(hall)
    gnode = _dot3(hsp, wsp(wn_ref))
    gf = gnode[:, :G3] + bhf0_ref[0:1]
    gr = gnode[:, G3:] + bhr0_ref[0:1]
    hf1 = _comb(gf, jnp.broadcast_to(bhf0_ref[1:2], (KB * NP, G3)), None)
    hr2 = _comb(gr, jnp.broadcast_to(bhr0_ref[1:2], (KB * NP, G3)), None)

    td = _sp(jnp.concatenate(
        [jnp.concatenate([hall[b * NP:(b + 1) * NP], hf1[b * NP:(b + 1) * NP]],
                         axis=1) for b in range(KB)], axis=1))
    ts = _sp(jnp.concatenate(
        [jnp.concatenate([hr2[b * NP:(b + 1) * NP], hall[b * NP:(b + 1) * NP]],
                         axis=1) for b in range(KB)], axis=1))

    logit_parts = []
    hs_parts = []
    sd_masks = []
    R = KB * TE
    for c in range(NCHUNK):
        sl = slice(c * TE, (c + 1) * TE)
        dst_c = dst[:, sl]
        src_c = src[:, sl]
        iota_c = jax.lax.broadcasted_iota(jnp.int32, (NP, TE), 0)
        sd_c = (iota_c == dst_c).astype(BF)
        ss_c = (iota_c == src_c).astype(BF)
        sd_masks.append(sd_c)
        gd = _dot2(sd_c, td, 0, 0)
        gs = _dot2(ss_c, ts, 0, 0)
        hd = jnp.concatenate([gd[:, b * 2 * H:b * 2 * H + H]
                              for b in range(KB)], axis=0)
        f1d = jnp.concatenate([gd[:, b * 2 * H + H:(b + 1) * 2 * H]
                               for b in range(KB)], axis=0)
        r2s = jnp.concatenate([gs[:, b * 2 * H:b * 2 * H + H]
                               for b in range(KB)], axis=0)
        hs = jnp.concatenate([gs[:, b * 2 * H + H:(b + 1) * 2 * H]
                              for b in range(KB)], axis=0)

        po1 = _dot3(_sp(jnp.concatenate([hd, r2s], axis=1)),
                    wsp(wa1_ref)) + ba1_ref[...]
        hr1 = _combP(po1, r2s)
        po2 = _dot3(_sp(jnp.concatenate([hs, f1d], axis=1)),
                    wsp(wa2_ref)) + ba2_ref[...]
        hf2 = _combP(po2, f1d)

        gi1 = _dot3(_sp(jnp.concatenate([hf2, r2s], axis=1)),
                    wsp(wb1_ref)) + bb1_ref[...]
        hr2l1 = _comb(gi1, jnp.broadcast_to(bhr1_ref[...], (R, G3)), None)
        gif = _dot3(_sp(jnp.concatenate([f1d, hr1], axis=1)),
                    wsp(wb2_ref)) + bb2_ref[...]
        m7 = _dot3(_sp(hr2l1), wsp(wc1_ref)) + bc1_ref[...]
        o_r = _comb(gif[:, :G3], m7, hr2l1)
        o_f = _comb(gif[:, G3:], jnp.broadcast_to(bhf1_ref[...], (R, G3)),
                    None)

        x = (_dot3(wsp(wc_ref),
                   _sp(jnp.concatenate([o_f, o_r], axis=1)), 1, 1)
             + bc_ref[0, 0])
        logit_parts.append(_gelu(x).reshape(KB, TE))
        hs_parts.append(hs)

    logit = jnp.concatenate(logit_parts, axis=1)
    lanes = jax.lax.broadcasted_iota(jnp.int32, (1, EP), 1)
    valid = lanes < E
    logit = jnp.where(valid, logit, np.float32(-1e30))

    iota_full = jax.lax.broadcasted_iota(jnp.int32, (NP, EP), 0)
    bs = iota_full == src

    def _seg_sum(row):
        return jnp.sum(jnp.where(bs, row, 0.0), axis=1, keepdims=True)

    def _seg_gather(col):
        return jnp.sum(jnp.where(bs, jnp.broadcast_to(col, (NP, EP)), 0.0),
                       axis=0, keepdims=True)

    zh = (logit + g) * np.float32(1.0 / TAU)
    es_all = jnp.where(valid, jnp.exp(logit), 0.0)
    coef_rows = []
    for b in range(KB):
        zb = zh[b:b + 1]
        m = jnp.max(jnp.where(bs, zb, np.float32(-1e30)), axis=1,
                    keepdims=True)
        eh = jnp.exp(zb - _seg_gather(m))
        eh = jnp.where(valid, eh, 0.0)
        es = es_all[b:b + 1]
        den = _seg_gather(_seg_sum(eh) * _seg_sum(es))
        coef_rows.append(jnp.where(valid,
                                   eh * es / (den + np.float32(1e-12)), 0.0))

    acc = jnp.zeros((NP, KB * H), F32)
    for c in range(NCHUNK):
        sl = slice(c * TE, (c + 1) * TE)
        msg = jnp.concatenate(
            [hs_parts[c][b * TE:(b + 1) * TE]
             * coef_rows[b][:, sl].reshape(TE, 1) for b in range(KB)],
            axis=1)
        acc = acc + _dot2(sd_masks[c], _sp(msg), 1, 0)
    for b in range(KB):
        out_ref[b] = acc[:, b * H:(b + 1) * H]


def _splitw(w):
    wh = w.astype(BF)
    wl = (w - wh.astype(F32)).astype(BF)
    return jnp.stack([wh, wl])


def kernel(h, params, edge_index):
    f32 = jnp.float32
    p0, p1 = params['l0'], params['l1']
    a1 = p1['W_ih_r'][:, :H]
    a2 = p1['W_ih_r'][:, H:]
    f1 = p1['W_ih_f'][:, :H]
    f2 = p1['W_ih_f'][:, H:]

    wn = _splitw(jnp.concatenate([p0['W_ih_f'].T, p0['W_ih_r'].T], axis=1))
    bhf0 = jnp.stack([p0['b_ih_f'], p0['b_hh_f']], axis=0)
    bhr0 = jnp.stack([p0['b_ih_r'], p0['b_hh_r']], axis=0)

    zH = jnp.zeros((H, H), f32)

    def _packA(wih, whh, bih, bhh):
        top = jnp.concatenate([wih.T[:, :2 * H], wih.T[:, 2 * H:], zH],
                              axis=1)
        bot = jnp.concatenate([whh.T[:, :2 * H], zH, whh.T[:, 2 * H:]],
                              axis=1)
        w = jnp.concatenate([top, bot], axis=0)
        b = jnp.concatenate([(bih + bhh)[:2 * H], bih[2 * H:], bhh[2 * H:]])
        return _splitw(w), b[None, :]

    wa1, ba1 = _packA(p0['W_ih_r'], p0['W_hh_r'],
                      p0['b_ih_r'], p0['b_hh_r'])
    wa2, ba2 = _packA(p0['W_ih_f'], p0['W_hh_f'],
                      p0['b_ih_f'], p0['b_hh_f'])
    wb1 = _splitw(jnp.concatenate([a1.T, a2.T], axis=0))
    bb1 = p1['b_ih_r'][None, :]
    wb2 = _splitw(jnp.concatenate(
        [jnp.concatenate([a1.T, a2.T], axis=0),
         jnp.concatenate([f1.T, f2.T], axis=0)], axis=1))
    bb2 = jnp.concatenate([p1['b_ih_r'], p1['b_ih_f']])[None, :]
    wc1 = _splitw(p1['W_hh_r'].T)
    bc1 = p1['b_hh_r'][None, :]
    bhr1 = p1['b_hh_r'][None, :]
    bhf1 = p1['b_hh_f'][None, :]
    wc = _splitw(params['Wc'][0:1, :])
    bc = params['bc'][None, :]

    u = jax.random.uniform(jax.random.key(42), (E, B),
                           minval=1e-6, maxval=1.0 - 1e-6)
    g = -jnp.log(-jnp.log(u))
    gp = jnp.zeros((B, 1, EP), f32).at[:, 0, :E].set(jnp.transpose(g))

    pad = jnp.full((1, EP - E), N, jnp.int32)
    dstr = jnp.concatenate([edge_index[0][None, :], pad], axis=1)
    srcr = jnp.concatenate([edge_index[1][None, :], pad], axis=1)

    hp = jnp.zeros((B, NP, H), f32).at[:, :N, :].set(h)

    full = lambda shape: pl.BlockSpec(shape, lambda b: (0,) * len(shape))
    grid_spec = pl.GridSpec(
        grid=(B // KB,),
        in_specs=[
            pl.BlockSpec((KB, NP, H), lambda b: (b, 0, 0)),
            full((1, EP)), full((1, EP)),
            pl.BlockSpec((KB, 1, EP), lambda b: (b, 0, 0)),
            full((2, H, 2 * G3)), full((2, G3)), full((2, G3)),
            full((2, 2 * H, 4 * H)), full((1, 4 * H)),
            full((2, 2 * H, 4 * H)), full((1, 4 * H)),
            full((2, 2 * H, G3)), full((1, G3)),
            full((2, 2 * H, 2 * G3)), full((1, 2 * G3)),
            full((2, H, G3)), full((1, G3)),
            full((1, G3)), full((1, G3)),
            full((2, 1, 2 * H)), full((1, 1)),
        ],
        out_specs=pl.BlockSpec((KB, NP, H), lambda b: (b, 0, 0)),
    )
    out = pl.pallas_call(
        _body,
        grid_spec=grid_spec,
        out_shape=jax.ShapeDtypeStruct((B, NP, H), f32),
    )(hp, dstr, srcr, gp,
      wn, bhf0, bhr0,
      wa1, ba1, wa2, ba2,
      wb1, bb1, wb2, bb2,
      wc1, bc1,
      bhr1, bhf1,
      wc, bc)
    return out[:, :N, :]
